# Initial kernel scaffold; baseline (speedup 1.0000x reference)
import jax
import jax.numpy as jnp
from jax.experimental import pallas as pl


def kernel(x, weight):
    raise NotImplementedError("write your pallas kernel here")


# speedup vs baseline: 2.2418x; 2.2418x over previous
import functools

import jax
import jax.numpy as jnp
from jax import lax
from jax.experimental import pallas as pl
from jax.experimental.pallas import tpu as pltpu
from jax.experimental.pallas import tpu_sc as plsc

NUM_EMB = 8192
DIM = 1024
B = 4 * 8192

_info = plsc.get_sparse_core_info()
_NC = _info.num_cores
_NS = _info.num_subcores
NW = _NC * _NS
BPW = B // NW
C = 32
NCHUNK = BPW // C


def _emb_body(x_hbm, w_hbm, out_hbm, idx_v, buf0, buf1, sg0, sg1, so0, so1):
    wid = lax.axis_index("s") * _NC + lax.axis_index("c")
    base = wid * BPW
    pltpu.sync_copy(x_hbm.at[pl.ds(base, BPW)], idx_v)

    bufs = (buf0, buf1)
    sgs = (sg0, sg1)
    sos = (so0, so1)

    def gather_start(g, b):
        pltpu.make_async_copy(
            w_hbm.at[idx_v.at[pl.ds(g * C, C)]], bufs[b], sgs[b]
        ).start()

    def gather_wait(b):
        pltpu.make_async_copy(
            w_hbm.at[idx_v.at[pl.ds(0, C)]], bufs[b], sgs[b]
        ).wait()

    def put_start(g, b):
        pltpu.make_async_copy(
            bufs[b], out_hbm.at[pl.ds(base + g * C, C)], sos[b]
        ).start()

    def put_wait(b):
        pltpu.make_async_copy(
            bufs[b], out_hbm.at[pl.ds(base, C)], sos[b]
        ).wait()

    gather_start(0, 0)
    for g in range(NCHUNK):
        b = g % 2
        gather_wait(b)
        nxt = g + 1
        if nxt < NCHUNK:
            if g >= 1:
                put_wait(nxt % 2)
            gather_start(nxt, nxt % 2)
        put_start(g, b)
    put_wait((NCHUNK - 2) % 2)
    put_wait((NCHUNK - 1) % 2)


@jax.jit
def _emb(x_flat, weight):
    mesh = plsc.VectorSubcoreMesh(core_axis_name="c", subcore_axis_name="s")
    fn = functools.partial(
        pl.kernel,
        mesh=mesh,
        out_type=jax.ShapeDtypeStruct((B, DIM), jnp.float32),
        scratch_types=[
            pltpu.VMEM((BPW,), jnp.int32),
            pltpu.VMEM((C, DIM), jnp.float32),
            pltpu.VMEM((C, DIM), jnp.float32),
            pltpu.SemaphoreType.DMA,
            pltpu.SemaphoreType.DMA,
            pltpu.SemaphoreType.DMA,
            pltpu.SemaphoreType.DMA,
        ],
    )(_emb_body)
    return fn(x_flat, weight)


def kernel(x, weight):
    out = _emb(x.reshape(-1), weight)
    return out.reshape(x.shape + (weight.shape[1],))


# speedup vs baseline: 2.3010x; 1.0264x over previous
import functools

import jax
import jax.numpy as jnp
from jax import lax
from jax.experimental import pallas as pl
from jax.experimental.pallas import tpu as pltpu
from jax.experimental.pallas import tpu_sc as plsc

NUM_EMB = 8192
DIM = 1024
B = 4 * 8192

_info = plsc.get_sparse_core_info()
_NC = _info.num_cores
_NS = _info.num_subcores
NW = _NC * _NS
BPW = B // NW
C = 32
NCHUNK = BPW // C


def _emb_body(x_hbm, w_hbm, out_hbm, idx_v, buf0, buf1, buf2,
              sg0, sg1, sg2, so0, so1, so2):
    wid = lax.axis_index("s") * _NC + lax.axis_index("c")
    base = wid * BPW
    pltpu.sync_copy(x_hbm.at[pl.ds(base, BPW)], idx_v)

    bufs = (buf0, buf1, buf2)
    sgs = (sg0, sg1, sg2)
    sos = (so0, so1, so2)
    NB = 3

    def gather_start(g, b):
        pltpu.make_async_copy(
            w_hbm.at[idx_v.at[pl.ds(g * C, C)]], bufs[b], sgs[b]
        ).start()

    def gather_wait(b):
        pltpu.make_async_copy(
            w_hbm.at[idx_v.at[pl.ds(0, C)]], bufs[b], sgs[b]
        ).wait()

    def put_start(g, b):
        pltpu.make_async_copy(
            bufs[b], out_hbm.at[pl.ds(base + g * C, C)], sos[b]
        ).start()

    def put_wait(b):
        pltpu.make_async_copy(
            bufs[b], out_hbm.at[pl.ds(base, C)], sos[b]
        ).wait()

    gather_start(0, 0)
    gather_start(1, 1)
    for g in range(NCHUNK):
        b = g % NB
        gather_wait(b)
        put_start(g, b)
        h = g + 2
        if h < NCHUNK:
            if h >= NB:
                put_wait(h % NB)
            gather_start(h, h % NB)
    for k in range(NCHUNK - 3, NCHUNK):
        put_wait(k % NB)


@jax.jit
def _emb(x_flat, weight):
    mesh = plsc.VectorSubcoreMesh(core_axis_name="c", subcore_axis_name="s")
    fn = functools.partial(
        pl.kernel,
        mesh=mesh,
        out_type=jax.ShapeDtypeStruct((B, DIM), jnp.float32),
        scratch_types=[
            pltpu.VMEM((BPW,), jnp.int32),
            pltpu.VMEM((C, DIM), jnp.float32),
            pltpu.VMEM((C, DIM), jnp.float32),
            pltpu.VMEM((C, DIM), jnp.float32),
            pltpu.SemaphoreType.DMA,
            pltpu.SemaphoreType.DMA,
            pltpu.SemaphoreType.DMA,
            pltpu.SemaphoreType.DMA,
            pltpu.SemaphoreType.DMA,
            pltpu.SemaphoreType.DMA,
        ],
    )(_emb_body)
    return fn(x_flat, weight)


def kernel(x, weight):
    out = _emb(x.reshape(-1), weight)
    return out.reshape(x.shape + (weight.shape[1],))


# speedup vs baseline: 2.3100x; 1.0039x over previous
import functools

import jax
import jax.numpy as jnp
from jax import lax
from jax.experimental import pallas as pl
from jax.experimental.pallas import tpu as pltpu
from jax.experimental.pallas import tpu_sc as plsc

NUM_EMB = 8192
DIM = 1024
B = 4 * 8192

_info = plsc.get_sparse_core_info()
_NC = _info.num_cores
_NS = _info.num_subcores
NW = _NC * _NS
BPW = B // NW
C = 40
_SIZES = [C] * (BPW // C) + ([BPW % C] if BPW % C else [])
_OFFS = [sum(_SIZES[:i]) for i in range(len(_SIZES))]
NCHUNK = len(_SIZES)


def _emb_body(x_hbm, w_hbm, out_hbm, idx_v, buf0, buf1, buf2,
              sg0, sg1, sg2, so0, so1, so2):
    wid = lax.axis_index("s") * _NC + lax.axis_index("c")
    base = wid * BPW
    pltpu.sync_copy(x_hbm.at[pl.ds(base, BPW)], idx_v)

    bufs = (buf0, buf1, buf2)
    sgs = (sg0, sg1, sg2)
    sos = (so0, so1, so2)
    NB = 3

    last_put = [None, None, None]

    def gather_start(g, b):
        n = _SIZES[g]
        pltpu.make_async_copy(
            w_hbm.at[idx_v.at[pl.ds(_OFFS[g], n)]],
            bufs[b].at[pl.ds(0, n)], sgs[b]
        ).start()

    def gather_wait(g, b):
        n = _SIZES[g]
        pltpu.make_async_copy(
            w_hbm.at[idx_v.at[pl.ds(0, n)]],
            bufs[b].at[pl.ds(0, n)], sgs[b]
        ).wait()

    def put_start(g, b):
        n = _SIZES[g]
        pltpu.make_async_copy(
            bufs[b].at[pl.ds(0, n)],
            out_hbm.at[pl.ds(base + _OFFS[g], n)], sos[b]
        ).start()
        last_put[b] = g

    def put_wait(b):
        n = _SIZES[last_put[b]]
        pltpu.make_async_copy(
            bufs[b].at[pl.ds(0, n)],
            out_hbm.at[pl.ds(base, n)], sos[b]
        ).wait()

    gather_start(0, 0)
    gather_start(1, 1)
    for g in range(NCHUNK):
        b = g % NB
        gather_wait(g, b)
        put_start(g, b)
        h = g + 2
        if h < NCHUNK:
            if h >= NB:
                put_wait(h % NB)
            gather_start(h, h % NB)
    for k in range(NCHUNK - 3, NCHUNK):
        put_wait(k % NB)


@jax.jit
def _emb(x_flat, weight):
    mesh = plsc.VectorSubcoreMesh(core_axis_name="c", subcore_axis_name="s")
    fn = functools.partial(
        pl.kernel,
        mesh=mesh,
        out_type=jax.ShapeDtypeStruct((B, DIM), jnp.float32),
        scratch_types=[
            pltpu.VMEM((BPW,), jnp.int32),
            pltpu.VMEM((C, DIM), jnp.float32),
            pltpu.VMEM((C, DIM), jnp.float32),
            pltpu.VMEM((C, DIM), jnp.float32),
            pltpu.SemaphoreType.DMA,
            pltpu.SemaphoreType.DMA,
            pltpu.SemaphoreType.DMA,
            pltpu.SemaphoreType.DMA,
            pltpu.SemaphoreType.DMA,
            pltpu.SemaphoreType.DMA,
        ],
    )(_emb_body)
    return fn(x_flat, weight)


def kernel(x, weight):
    out = _emb(x.reshape(-1), weight)
    return out.reshape(x.shape + (weight.shape[1],))


# speedup vs baseline: 2.3102x; 1.0001x over previous
import functools

import jax
import jax.numpy as jnp
from jax import lax
from jax.experimental import pallas as pl
from jax.experimental.pallas import tpu as pltpu
from jax.experimental.pallas import tpu_sc as plsc

NUM_EMB = 8192
DIM = 1024
B = 4 * 8192

_info = plsc.get_sparse_core_info()
_NC = _info.num_cores
_NS = _info.num_subcores
NW = _NC * _NS
BPW = B // NW
C = 40
_SIZES = [C] * (BPW // C) + ([BPW % C] if BPW % C else [])
_OFFS = [sum(_SIZES[:i]) for i in range(len(_SIZES))]
NCHUNK = len(_SIZES)


def _emb_body(x_hbm, w_hbm, out_hbm, idx_v, buf0, buf1, buf2,
              sg0, sg1, sg2, so0, so1, so2):
    wid = lax.axis_index("s") * _NC + lax.axis_index("c")
    base = wid * BPW
    _head = 2 * C
    pltpu.sync_copy(x_hbm.at[pl.ds(base, _head)], idx_v.at[pl.ds(0, _head)])

    bufs = (buf0, buf1, buf2)
    sgs = (sg0, sg1, sg2)
    sos = (so0, so1, so2)
    NB = 3

    last_put = [None, None, None]

    def gather_start(g, b):
        n = _SIZES[g]
        pltpu.make_async_copy(
            w_hbm.at[idx_v.at[pl.ds(_OFFS[g], n)]],
            bufs[b].at[pl.ds(0, n)], sgs[b]
        ).start()

    def gather_wait(g, b):
        n = _SIZES[g]
        pltpu.make_async_copy(
            w_hbm.at[idx_v.at[pl.ds(0, n)]],
            bufs[b].at[pl.ds(0, n)], sgs[b]
        ).wait()

    def put_start(g, b):
        n = _SIZES[g]
        pltpu.make_async_copy(
            bufs[b].at[pl.ds(0, n)],
            out_hbm.at[pl.ds(base + _OFFS[g], n)], sos[b]
        ).start()
        last_put[b] = g

    def put_wait(b):
        n = _SIZES[last_put[b]]
        pltpu.make_async_copy(
            bufs[b].at[pl.ds(0, n)],
            out_hbm.at[pl.ds(base, n)], sos[b]
        ).wait()

    gather_start(0, 0)
    gather_start(1, 1)
    pltpu.sync_copy(
        x_hbm.at[pl.ds(base + _head, BPW - _head)],
        idx_v.at[pl.ds(_head, BPW - _head)],
    )
    for g in range(NCHUNK):
        b = g % NB
        gather_wait(g, b)
        put_start(g, b)
        h = g + 2
        if h < NCHUNK:
            if h >= NB:
                put_wait(h % NB)
            gather_start(h, h % NB)
    for k in range(NCHUNK - 3, NCHUNK):
        put_wait(k % NB)


@jax.jit
def _emb(x_flat, weight):
    mesh = plsc.VectorSubcoreMesh(core_axis_name="c", subcore_axis_name="s")
    fn = functools.partial(
        pl.kernel,
        mesh=mesh,
        out_type=jax.ShapeDtypeStruct((B, DIM), jnp.float32),
        scratch_types=[
            pltpu.VMEM((BPW,), jnp.int32),
            pltpu.VMEM((C, DIM), jnp.float32),
            pltpu.VMEM((C, DIM), jnp.float32),
            pltpu.VMEM((C, DIM), jnp.float32),
            pltpu.SemaphoreType.DMA,
            pltpu.SemaphoreType.DMA,
            pltpu.SemaphoreType.DMA,
            pltpu.SemaphoreType.DMA,
            pltpu.SemaphoreType.DMA,
            pltpu.SemaphoreType.DMA,
        ],
    )(_emb_body)
    return fn(x_flat, weight)


def kernel(x, weight):
    out = _emb(x.reshape(-1), weight)
    return out.reshape(x.shape + (weight.shape[1],))
